# baseline (speedup 1.0000x reference)
import functools

import jax
import jax.numpy as jnp
from jax.experimental import pallas as pl
from jax.experimental.pallas import tpu as pltpu

_MAIN = 2560
_WARMUP = (512, 1024)
_NBUF = 3


def _schedule(k_dim):
    chunks = list(_WARMUP)
    rem = k_dim - sum(_WARMUP)
    n_main = rem // _MAIN
    chunks += [_MAIN] * n_main
    rem -= n_main * _MAIN
    if rem:
        aligned = (rem // 128) * 128
        if aligned:
            chunks.append(aligned)
        if rem - aligned:
            chunks.append(rem - aligned)
    offs, o = [], 0
    for c in chunks:
        offs.append(o)
        o += c
    return chunks, offs


def _mm_kernel(bt_ref, at_hbm, o_ref, abuf, sems, *, chunks, offs):
    n, m = o_ref.shape

    def copy(i):
        return pltpu.make_async_copy(
            at_hbm.at[pl.ds(offs[i], chunks[i]), :],
            abuf.at[i % _NBUF, pl.ds(0, chunks[i]), :],
            sems.at[i % _NBUF],
        )

    for i in range(_NBUF - 1):
        copy(i).start()

    acc = jnp.zeros((n, m), jnp.float32)
    for i in range(len(chunks)):
        if i + _NBUF - 1 < len(chunks):
            copy(i + _NBUF - 1).start()
        copy(i).wait()
        bt = bt_ref[:, pl.ds(offs[i], chunks[i])]
        a = abuf[i % _NBUF, pl.ds(0, chunks[i]), :]
        acc += jnp.dot(
            bt.astype(jnp.bfloat16),
            a.astype(jnp.bfloat16),
            preferred_element_type=jnp.float32,
        )
    o_ref[...] = acc


def kernel(attention_weights, content_matrix):
    m, k_dim = attention_weights.shape
    _, n = content_matrix.shape
    at = attention_weights.T
    bt = content_matrix.T
    chunks, offs = _schedule(k_dim)
    body = functools.partial(_mm_kernel, chunks=chunks, offs=offs)
    out_t = pl.pallas_call(
        body,
        grid=(1,),
        in_specs=[
            pl.BlockSpec((n, k_dim), lambda i: (0, 0)),
            pl.BlockSpec(memory_space=pltpu.MemorySpace.HBM),
        ],
        out_specs=pl.BlockSpec((n, m), lambda i: (0, 0)),
        out_shape=jax.ShapeDtypeStruct((n, m), jnp.float32),
        scratch_shapes=[
            pltpu.VMEM((_NBUF, _MAIN, m), jnp.float32),
            pltpu.SemaphoreType.DMA((_NBUF,)),
        ],
    )(bt, at)
    return out_t.T


# speedup vs baseline: 1.0297x; 1.0297x over previous
import functools

import jax
import jax.numpy as jnp
from jax import lax
from jax.experimental import pallas as pl
from jax.experimental.pallas import tpu as pltpu

_BLK_K = 2816


def _mm_kernel(bt_ref, at_ref, o_ref, *, nsteps, tail):
    k = pl.program_id(0)

    @pl.when(k == 0)
    def _init():
        o_ref[...] = jnp.zeros_like(o_ref)

    @pl.when(k < nsteps - 1)
    def _body():
        o_ref[...] += jnp.dot(
            bt_ref[...].astype(jnp.bfloat16),
            at_ref[...].astype(jnp.bfloat16),
            preferred_element_type=jnp.float32,
        )

    @pl.when(k == nsteps - 1)
    def _tail():
        bt = bt_ref[...]
        col = lax.broadcasted_iota(jnp.int32, bt.shape, 1)
        bt = jnp.where(col < tail, bt, 0.0)
        at = at_ref[...]
        row = lax.broadcasted_iota(jnp.int32, at.shape, 0)
        at = jnp.where(row < tail, at, 0.0)
        o_ref[...] += jnp.dot(
            bt.astype(jnp.bfloat16),
            at.astype(jnp.bfloat16),
            preferred_element_type=jnp.float32,
        )


def kernel(attention_weights, content_matrix):
    m, k_dim = attention_weights.shape
    _, n = content_matrix.shape
    at = attention_weights.T
    bt = content_matrix.T
    nsteps = pl.cdiv(k_dim, _BLK_K)
    tail = k_dim - (nsteps - 1) * _BLK_K
    body = functools.partial(_mm_kernel, nsteps=nsteps, tail=tail)
    out_t = pl.pallas_call(
        body,
        grid=(nsteps,),
        in_specs=[
            pl.BlockSpec((n, _BLK_K), lambda k: (0, k)),
            pl.BlockSpec((_BLK_K, m), lambda k: (k, 0)),
        ],
        out_specs=pl.BlockSpec((n, m), lambda k: (0, 0)),
        out_shape=jax.ShapeDtypeStruct((n, m), jnp.float32),
        compiler_params=pltpu.CompilerParams(
            dimension_semantics=("arbitrary",)
        ),
    )(bt, at)
    return out_t.T
